# baseline (speedup 1.0000x reference)
import functools

import jax
import jax.numpy as jnp
from jax import lax
from jax.experimental import pallas as pl
from jax.experimental.pallas import tpu as pltpu
from jax.experimental.pallas import tpu_sc as plsc

N = 10000
E = 160000
D = 256
NPAD = 10240
NC = 2
NS = 16
ROWS_PER_TILE = NPAD // NS
EDGES_PER_TILE = E // NS
CHUNK = 128
NCHUNK = 80
EPAD_TILE = NCHUNK * CHUNK
NPHASE = 2
HALF = NCHUNK // NPHASE
NB = 2
ROUNDS = HALF // NB
EMB_CHUNK = 80
EMB_NCHUNK = 4
GRID = NPAD // 256

_f32 = jnp.float32


def _emb_deg_body(nid_hbm, srcp_hbm, dstp_hbm, emb_hbm, ones_hbm, zeros1_hbm,
             h0_out, degout_out, degin_out,
             nidx_v, rows_v, eidx_v, ones_v, deg_sh, sem):
  c = lax.axis_index("c")
  s = lax.axis_index("s")
  w = c * NS + s

  pltpu.sync_copy(zeros1_hbm, deg_sh.at[pl.ds(s * ROWS_PER_TILE, ROWS_PER_TILE)])
  pltpu.sync_copy(ones_hbm, ones_v)

  pltpu.sync_copy(nid_hbm.at[w], nidx_v)

  def emb_chunk(j, carry):
    pltpu.async_copy(emb_hbm.at[nidx_v.at[j]], rows_v, sem).wait()
    base = pl.multiple_of(w * (EMB_NCHUNK * EMB_CHUNK) + j * EMB_CHUNK, 8)
    pltpu.sync_copy(rows_v, h0_out.at[pl.ds(base, EMB_CHUNK)])
    return carry

  lax.fori_loop(0, EMB_NCHUNK, emb_chunk, 0)

  plsc.subcore_barrier()

  def deg_pass(edges_hbm):
    pltpu.sync_copy(edges_hbm.at[s], eidx_v)

    def deg_chunk(j, carry):
      pltpu.sync_copy(ones_v, deg_sh.at[eidx_v.at[j]], add=True)
      return carry

    lax.fori_loop(0, NCHUNK, deg_chunk, 0)

  @pl.when(c == 0)
  def _():
    deg_pass(srcp_hbm)

  @pl.when(c == 1)
  def _():
    deg_pass(dstp_hbm)

  plsc.subcore_barrier()

  sl = pl.ds(s * ROWS_PER_TILE, ROWS_PER_TILE)

  @pl.when(c == 0)
  def _():
    pltpu.sync_copy(deg_sh.at[sl], degout_out.at[sl])

  @pl.when(c == 1)
  def _():
    pltpu.sync_copy(deg_sh.at[sl], degin_out.at[sl])


def _mp_body(gl_hbm, gr_hbm, srcp_hbm, dstp_hbm, zeros_hbm,
             outl, outr, sidx_v, didx_v, rows_v, acc_sh, *sems):
  gs = list(sems[:NB])
  ss = list(sems[NB:])
  c = lax.axis_index("c")
  s = lax.axis_index("s")
  sl = pl.ds(s * ROWS_PER_TILE, ROWS_PER_TILE)

  pltpu.sync_copy(zeros_hbm, acc_sh.at[sl])
  plsc.subcore_barrier()

  def run(g_hbm):
    for p in range(NPHASE):
      pltpu.sync_copy(srcp_hbm.at[s, pl.ds(p * HALF, HALF)], sidx_v)
      pltpu.sync_copy(dstp_hbm.at[s, pl.ds(p * HALF, HALF)], didx_v)
      def gbuf(b):
        return rows_v.at[pl.ds(b * CHUNK, CHUNK)]

      def fire_gather(b, j):
        pltpu.async_copy(g_hbm.at[sidx_v.at[j]], gbuf(b), gs[b])

      def fire_scatter(b, j):
        pltpu.async_copy(gbuf(b), acc_sh.at[didx_v.at[j]], ss[b], add=True)

      def wait_gather(b):
        pltpu.make_async_copy(g_hbm.at[pl.ds(0, CHUNK)], gbuf(b),
                              gs[b]).wait()

      def wait_scatter(b):
        pltpu.make_async_copy(gbuf(b), acc_sh.at[pl.ds(0, CHUNK)],
                              ss[b]).wait()

      for b in range(NB):
        fire_gather(b, b)

      def ring(r, carry):
        j0 = r * NB
        for b in range(NB):
          wait_gather(b)
          fire_scatter(b, j0 + b)

        @pl.when(r < ROUNDS - 1)
        def _():
          for b in range(NB):
            wait_scatter(b)
            fire_gather(b, j0 + NB + b)

        return carry

      lax.fori_loop(0, ROUNDS, ring, 0)
      for b in range(NB):
        wait_scatter(b)

  @pl.when(c == 0)
  def _():
    run(gl_hbm)

  @pl.when(c == 1)
  def _():
    run(gr_hbm)

  plsc.subcore_barrier()

  @pl.when(c == 0)
  def _():
    pltpu.sync_copy(acc_sh.at[sl], outl.at[sl])

  @pl.when(c == 1)
  def _():
    pltpu.sync_copy(acc_sh.at[sl], outr.at[sl])


def _pre_body(h0_ref, doutb_ref, gl_ref, gr_ref):
  deg = doutb_ref[...]
  norm = jnp.where(deg > 0, lax.rsqrt(deg), 0.0)
  h = h0_ref[...]
  gl_ref[...] = h[:, :128] * norm
  gr_ref[...] = h[:, 128:] * norm


def _mm_body(al_ref, ar_ref, dinb_ref, doutb_ref, w_ref, b_ref,
             gl_ref, gr_ref):
  din = dinb_ref[...]
  nin = jnp.where(din > 0, lax.rsqrt(din), 0.0)
  x = jnp.concatenate([al_ref[...] * nin, ar_ref[...] * nin], axis=1)
  h = jnp.dot(x, w_ref[...], preferred_element_type=jnp.float32) + b_ref[...]
  dout = doutb_ref[...]
  nout = jnp.where(dout > 0, lax.rsqrt(dout), 0.0)
  gl_ref[...] = h[:, :128] * nout
  gr_ref[...] = h[:, 128:] * nout


def _mm2_body(al_ref, ar_ref, dinb_ref, w_ref, b_ref, lng_ref, lnb_ref,
              lw_ref, lb_ref, out_ref, max_scr):
  n = pl.program_id(0)
  din = dinb_ref[...]
  nin = jnp.where(din > 0, lax.rsqrt(din), 0.0)
  x = jnp.concatenate([al_ref[...] * nin, ar_ref[...] * nin], axis=1)
  h = jnp.dot(x, w_ref[...], preferred_element_type=jnp.float32) + b_ref[...]
  rid = n * 256 + lax.broadcasted_iota(jnp.int32, (256, D), 0)
  hm = jnp.where(rid < N, h, -jnp.inf)
  bmax = jnp.max(hm, axis=0, keepdims=True)

  @pl.when(n == 0)
  def _():
    max_scr[...] = bmax

  @pl.when(n > 0)
  def _():
    max_scr[...] = jnp.maximum(max_scr[...], bmax)

  @pl.when(n == GRID - 1)
  def _():
    pooled = max_scr[...]
    mu = jnp.mean(pooled)
    var = jnp.mean((pooled - mu) ** 2)
    xn = (pooled - mu) * lax.rsqrt(var + 1e-5) * lng_ref[...] + lnb_ref[...]
    o = jnp.dot(xn, lw_ref[...], preferred_element_type=jnp.float32)
    out_ref[...] = jnp.maximum(o + lb_ref[...], 0.0)


def _block(n128):
  return pl.BlockSpec((256, n128), lambda n: (n, 0))


def _full(shape):
  return pl.BlockSpec(shape, lambda n: tuple(0 for _ in shape))


_pre_call = pl.pallas_call(
    _pre_body,
    grid=(GRID,),
    in_specs=[_block(D), _block(128)],
    out_specs=[_block(128), _block(128)],
    out_shape=[jax.ShapeDtypeStruct((NPAD, 128), _f32)] * 2,
)

_mm1_call = pl.pallas_call(
    _mm_body,
    grid=(GRID,),
    in_specs=[_block(128), _block(128), _block(128), _block(128),
              _full((D, D)), _full((1, D))],
    out_specs=[_block(128), _block(128)],
    out_shape=[jax.ShapeDtypeStruct((NPAD, 128), _f32)] * 2,
)

_mm2_call = pl.pallas_call(
    _mm2_body,
    grid=(GRID,),
    in_specs=[_block(128), _block(128), _block(128),
              _full((D, D)), _full((1, D)), _full((1, D)), _full((1, D)),
              _full((D, 32)), _full((1, 32))],
    out_specs=pl.BlockSpec((1, 32), lambda n: (0, 0)),
    out_shape=jax.ShapeDtypeStruct((1, 32), _f32),
    scratch_shapes=[pltpu.VMEM((1, D), _f32)],
)


@functools.lru_cache(maxsize=1)
def _sc_kernels():
  mesh = plsc.VectorSubcoreMesh(core_axis_name="c", subcore_axis_name="s",
                                num_cores=NC, num_subcores=NS)
  emb_deg = pl.kernel(
      _emb_deg_body,
      out_type=[
          jax.ShapeDtypeStruct((NPAD, D), _f32),
          jax.ShapeDtypeStruct((NPAD,), _f32),
          jax.ShapeDtypeStruct((NPAD,), _f32),
      ],
      mesh=mesh,
      scratch_types=[
          pltpu.VMEM((EMB_NCHUNK, EMB_CHUNK), jnp.int32),
          pltpu.VMEM((EMB_CHUNK, D), _f32),
          pltpu.VMEM((NCHUNK, CHUNK), jnp.int32),
          pltpu.VMEM((CHUNK,), _f32),
          pltpu.VMEM_SHARED((NPAD,), _f32),
          pltpu.SemaphoreType.DMA,
      ],
  )
  mp = pl.kernel(
      _mp_body,
      out_type=[
          jax.ShapeDtypeStruct((NPAD, 128), _f32),
          jax.ShapeDtypeStruct((NPAD, 128), _f32),
      ],
      mesh=mesh,
      scratch_types=[
          pltpu.VMEM((HALF, CHUNK), jnp.int32),
          pltpu.VMEM((HALF, CHUNK), jnp.int32),
          pltpu.VMEM((NB * CHUNK, 128), _f32),
          pltpu.VMEM_SHARED((NPAD, 128), _f32),
      ] + [pltpu.SemaphoreType.DMA] * (2 * NB),
  )
  return emb_deg, mp


def _pad_edges(e):
  e = e.reshape(NS, EDGES_PER_TILE)
  pad = N + (jnp.arange(EPAD_TILE - EDGES_PER_TILE, dtype=jnp.int32) % (NPAD - N))
  pad = jnp.broadcast_to(pad, (NS, EPAD_TILE - EDGES_PER_TILE))
  return jnp.concatenate([e, pad], axis=1).reshape(NS, NCHUNK, CHUNK)


def kernel(node_ids, edge_index, emb_table, W1, b1, W2, b2,
           ln_gamma, ln_beta, lin_W, lin_b):
  node_ids = node_ids.astype(jnp.int32)
  src = edge_index[0].astype(jnp.int32)
  dst = edge_index[1].astype(jnp.int32)

  nid_pad = jnp.concatenate(
      [node_ids, jnp.zeros((NPAD - N,), jnp.int32)]
  ).reshape(NC * NS, EMB_NCHUNK, EMB_CHUNK)
  srcp = _pad_edges(src)
  dstp = _pad_edges(dst)

  ones_c = jnp.ones((CHUNK,), _f32)
  zeros1 = jnp.zeros((ROWS_PER_TILE,), _f32)
  zeros2 = jnp.zeros((ROWS_PER_TILE, 128), _f32)

  emb_deg, mp = _sc_kernels()
  h0, deg_out, deg_in = emb_deg(nid_pad, srcp, dstp, emb_table,
                                ones_c, zeros1)

  dout_b = jnp.broadcast_to(deg_out[:, None], (NPAD, 128))
  din_b = jnp.broadcast_to(deg_in[:, None], (NPAD, 128))

  g0l, g0r = _pre_call(h0, dout_b)
  a1l, a1r = mp(g0l, g0r, srcp, dstp, zeros2)
  g1l, g1r = _mm1_call(a1l, a1r, din_b, dout_b, W1, b1.reshape(1, D))
  a2l, a2r = mp(g1l, g1r, srcp, dstp, zeros2)
  out = _mm2_call(a2l, a2r, din_b, W2, b2.reshape(1, D),
                  ln_gamma.reshape(1, D), ln_beta.reshape(1, D),
                  lin_W, lin_b.reshape(1, 32))
  return out.reshape(32)


# speedup vs baseline: 1.1560x; 1.1560x over previous
import functools

import jax
import jax.numpy as jnp
from jax import lax
from jax.experimental import pallas as pl
from jax.experimental.pallas import tpu as pltpu
from jax.experimental.pallas import tpu_sc as plsc

N = 10000
E = 160000
D = 256
NPAD = 10240
NC = 2
NS = 16
ROWS_PER_TILE = NPAD // NS
EDGES_PER_TILE = E // NS
CHUNK = 64
IDXROW = 128
NROW = 80
NCHUNK = 160
EPAD_TILE = NROW * IDXROW
NB = 4
ROUNDS = NCHUNK // NB
EMB_CHUNK = 80
EMB_NCHUNK = 4
GRID = NPAD // 256

_f32 = jnp.float32


def _emb_deg_body(nid_hbm, srcp_hbm, dstp_hbm, emb_hbm, ones_hbm, zeros1_hbm,
             h0_out, degout_out, degin_out,
             nidx_v, rows_v, eidx_v, ones_v, deg_sh, sem):
  c = lax.axis_index("c")
  s = lax.axis_index("s")
  w = c * NS + s

  pltpu.sync_copy(zeros1_hbm, deg_sh.at[pl.ds(s * ROWS_PER_TILE, ROWS_PER_TILE)])
  pltpu.sync_copy(ones_hbm, ones_v)

  pltpu.sync_copy(nid_hbm.at[w], nidx_v)

  def emb_chunk(j, carry):
    pltpu.async_copy(emb_hbm.at[nidx_v.at[j]], rows_v, sem).wait()
    base = pl.multiple_of(w * (EMB_NCHUNK * EMB_CHUNK) + j * EMB_CHUNK, 8)
    pltpu.sync_copy(rows_v, h0_out.at[pl.ds(base, EMB_CHUNK)])
    return carry

  lax.fori_loop(0, EMB_NCHUNK, emb_chunk, 0)

  plsc.subcore_barrier()

  def deg_pass(edges_hbm):
    pltpu.sync_copy(edges_hbm.at[s], eidx_v)

    def deg_chunk(j, carry):
      pltpu.sync_copy(ones_v, deg_sh.at[eidx_v.at[j]], add=True)
      return carry

    lax.fori_loop(0, NROW, deg_chunk, 0)

  @pl.when(c == 0)
  def _():
    deg_pass(srcp_hbm)

  @pl.when(c == 1)
  def _():
    deg_pass(dstp_hbm)

  plsc.subcore_barrier()

  sl = pl.ds(s * ROWS_PER_TILE, ROWS_PER_TILE)

  @pl.when(c == 0)
  def _():
    pltpu.sync_copy(deg_sh.at[sl], degout_out.at[sl])

  @pl.when(c == 1)
  def _():
    pltpu.sync_copy(deg_sh.at[sl], degin_out.at[sl])


def _mp_body(gl_hbm, gr_hbm, srcp_hbm, dstp_hbm, zeros_hbm,
             outl, outr, sidx_v, didx_v, rows_v, acc_sh, *sems):
  gs = list(sems[:NB])
  ss = list(sems[NB:])
  c = lax.axis_index("c")
  s = lax.axis_index("s")
  sl = pl.ds(s * ROWS_PER_TILE, ROWS_PER_TILE)

  pltpu.sync_copy(zeros_hbm, acc_sh.at[sl])
  pltpu.sync_copy(srcp_hbm.at[s], sidx_v)
  plsc.subcore_barrier()

  def run(g_hbm):
    def idx_slice(idx_v, r, b):
      return idx_v.at[(NB // 2) * r + b // 2,
                      pl.ds((b % 2) * CHUNK, CHUNK)]

    def gbuf(b):
      return rows_v.at[pl.ds(b * CHUNK, CHUNK)]

    def fire_gather(b, r):
      pltpu.async_copy(g_hbm.at[idx_slice(sidx_v, r, b)], gbuf(b), gs[b])

    def fire_scatter(b, rl):
      pltpu.async_copy(gbuf(b), acc_sh.at[idx_slice(didx_v, rl, b)],
                       ss[b], add=True)

    def wait_gather(b):
      pltpu.make_async_copy(g_hbm.at[pl.ds(0, CHUNK)], gbuf(b),
                            gs[b]).wait()

    def wait_scatter(b):
      pltpu.make_async_copy(gbuf(b), acc_sh.at[pl.ds(0, CHUNK)],
                            ss[b]).wait()

    rh = ROUNDS // 2
    for h in range(2):
      pltpu.sync_copy(dstp_hbm.at[s, pl.ds(h * (NROW // 2), NROW // 2)],
                      didx_v)
      for b in range(NB):
        fire_gather(b, h * rh)

      def ring(rl, carry):
        for b in range(NB):
          wait_gather(b)
          fire_scatter(b, rl)

        @pl.when(rl < rh - 1)
        def _():
          for b in range(NB):
            wait_scatter(b)
            fire_gather(b, h * rh + rl + 1)

        return carry

      lax.fori_loop(0, rh, ring, 0)
      for b in range(NB):
        wait_scatter(b)

  @pl.when(c == 0)
  def _():
    run(gl_hbm)

  @pl.when(c == 1)
  def _():
    run(gr_hbm)

  plsc.subcore_barrier()

  @pl.when(c == 0)
  def _():
    pltpu.sync_copy(acc_sh.at[sl], outl.at[sl])

  @pl.when(c == 1)
  def _():
    pltpu.sync_copy(acc_sh.at[sl], outr.at[sl])


def _pre_body(h0_ref, doutb_ref, gl_ref, gr_ref):
  deg = doutb_ref[...]
  norm = jnp.where(deg > 0, lax.rsqrt(deg), 0.0)
  h = h0_ref[...]
  gl_ref[...] = h[:, :128] * norm
  gr_ref[...] = h[:, 128:] * norm


def _mm_body(al_ref, ar_ref, dinb_ref, doutb_ref, w_ref, b_ref,
             gl_ref, gr_ref):
  din = dinb_ref[...]
  nin = jnp.where(din > 0, lax.rsqrt(din), 0.0)
  x = jnp.concatenate([al_ref[...] * nin, ar_ref[...] * nin], axis=1)
  h = jnp.dot(x, w_ref[...], preferred_element_type=jnp.float32) + b_ref[...]
  dout = doutb_ref[...]
  nout = jnp.where(dout > 0, lax.rsqrt(dout), 0.0)
  gl_ref[...] = h[:, :128] * nout
  gr_ref[...] = h[:, 128:] * nout


def _mm2_body(al_ref, ar_ref, dinb_ref, w_ref, b_ref, lng_ref, lnb_ref,
              lw_ref, lb_ref, out_ref, max_scr):
  n = pl.program_id(0)
  din = dinb_ref[...]
  nin = jnp.where(din > 0, lax.rsqrt(din), 0.0)
  x = jnp.concatenate([al_ref[...] * nin, ar_ref[...] * nin], axis=1)
  h = jnp.dot(x, w_ref[...], preferred_element_type=jnp.float32) + b_ref[...]
  rid = n * 256 + lax.broadcasted_iota(jnp.int32, (256, D), 0)
  hm = jnp.where(rid < N, h, -jnp.inf)
  bmax = jnp.max(hm, axis=0, keepdims=True)

  @pl.when(n == 0)
  def _():
    max_scr[...] = bmax

  @pl.when(n > 0)
  def _():
    max_scr[...] = jnp.maximum(max_scr[...], bmax)

  @pl.when(n == GRID - 1)
  def _():
    pooled = max_scr[...]
    mu = jnp.mean(pooled)
    var = jnp.mean((pooled - mu) ** 2)
    xn = (pooled - mu) * lax.rsqrt(var + 1e-5) * lng_ref[...] + lnb_ref[...]
    o = jnp.dot(xn, lw_ref[...], preferred_element_type=jnp.float32)
    out_ref[...] = jnp.maximum(o + lb_ref[...], 0.0)


def _block(n128):
  return pl.BlockSpec((256, n128), lambda n: (n, 0))


def _full(shape):
  return pl.BlockSpec(shape, lambda n: tuple(0 for _ in shape))


_pre_call = pl.pallas_call(
    _pre_body,
    grid=(GRID,),
    in_specs=[_block(D), _block(128)],
    out_specs=[_block(128), _block(128)],
    out_shape=[jax.ShapeDtypeStruct((NPAD, 128), _f32)] * 2,
)

_mm1_call = pl.pallas_call(
    _mm_body,
    grid=(GRID,),
    in_specs=[_block(128), _block(128), _block(128), _block(128),
              _full((D, D)), _full((1, D))],
    out_specs=[_block(128), _block(128)],
    out_shape=[jax.ShapeDtypeStruct((NPAD, 128), _f32)] * 2,
)

_mm2_call = pl.pallas_call(
    _mm2_body,
    grid=(GRID,),
    in_specs=[_block(128), _block(128), _block(128),
              _full((D, D)), _full((1, D)), _full((1, D)), _full((1, D)),
              _full((D, 32)), _full((1, 32))],
    out_specs=pl.BlockSpec((1, 32), lambda n: (0, 0)),
    out_shape=jax.ShapeDtypeStruct((1, 32), _f32),
    scratch_shapes=[pltpu.VMEM((1, D), _f32)],
)


@functools.lru_cache(maxsize=1)
def _sc_kernels():
  mesh = plsc.VectorSubcoreMesh(core_axis_name="c", subcore_axis_name="s",
                                num_cores=NC, num_subcores=NS)
  emb_deg = pl.kernel(
      _emb_deg_body,
      out_type=[
          jax.ShapeDtypeStruct((NPAD, D), _f32),
          jax.ShapeDtypeStruct((NPAD,), _f32),
          jax.ShapeDtypeStruct((NPAD,), _f32),
      ],
      mesh=mesh,
      scratch_types=[
          pltpu.VMEM((EMB_NCHUNK, EMB_CHUNK), jnp.int32),
          pltpu.VMEM((EMB_CHUNK, D), _f32),
          pltpu.VMEM((NROW, IDXROW), jnp.int32),
          pltpu.VMEM((IDXROW,), _f32),
          pltpu.VMEM_SHARED((NPAD,), _f32),
          pltpu.SemaphoreType.DMA,
      ],
  )
  mp = pl.kernel(
      _mp_body,
      out_type=[
          jax.ShapeDtypeStruct((NPAD, 128), _f32),
          jax.ShapeDtypeStruct((NPAD, 128), _f32),
      ],
      mesh=mesh,
      scratch_types=[
          pltpu.VMEM((NROW, IDXROW), jnp.int32),
          pltpu.VMEM((NROW // 2, IDXROW), jnp.int32),
          pltpu.VMEM((NB * CHUNK, 128), _f32),
          pltpu.VMEM_SHARED((NPAD, 128), _f32),
      ] + [pltpu.SemaphoreType.DMA] * (2 * NB),
  )
  return emb_deg, mp


def _pad_edges(e):
  e = e.reshape(NS, EDGES_PER_TILE)
  pad = N + (jnp.arange(EPAD_TILE - EDGES_PER_TILE, dtype=jnp.int32) % (NPAD - N))
  pad = jnp.broadcast_to(pad, (NS, EPAD_TILE - EDGES_PER_TILE))
  return jnp.concatenate([e, pad], axis=1).reshape(NS, NROW, IDXROW)


def kernel(node_ids, edge_index, emb_table, W1, b1, W2, b2,
           ln_gamma, ln_beta, lin_W, lin_b):
  node_ids = node_ids.astype(jnp.int32)
  src = edge_index[0].astype(jnp.int32)
  dst = edge_index[1].astype(jnp.int32)

  nid_pad = jnp.concatenate(
      [node_ids, jnp.zeros((NPAD - N,), jnp.int32)]
  ).reshape(NC * NS, EMB_NCHUNK, EMB_CHUNK)
  srcp = _pad_edges(src)
  dstp = _pad_edges(dst)

  ones_c = jnp.ones((IDXROW,), _f32)
  zeros1 = jnp.zeros((ROWS_PER_TILE,), _f32)
  zeros2 = jnp.zeros((ROWS_PER_TILE, 128), _f32)

  emb_deg, mp = _sc_kernels()
  h0, deg_out, deg_in = emb_deg(nid_pad, srcp, dstp, emb_table,
                                ones_c, zeros1)

  dout_b = jnp.broadcast_to(deg_out[:, None], (NPAD, 128))
  din_b = jnp.broadcast_to(deg_in[:, None], (NPAD, 128))

  g0l, g0r = _pre_call(h0, dout_b)
  a1l, a1r = mp(g0l, g0r, srcp, dstp, zeros2)
  g1l, g1r = _mm1_call(a1l, a1r, din_b, dout_b, W1, b1.reshape(1, D))
  a2l, a2r = mp(g1l, g1r, srcp, dstp, zeros2)
  out = _mm2_call(a2l, a2r, din_b, W2, b2.reshape(1, D),
                  ln_gamma.reshape(1, D), ln_beta.reshape(1, D),
                  lin_W, lin_b.reshape(1, 32))
  return out.reshape(32)


# speedup vs baseline: 1.1678x; 1.0102x over previous
import functools

import jax
import jax.numpy as jnp
from jax import lax
from jax.experimental import pallas as pl
from jax.experimental.pallas import tpu as pltpu
from jax.experimental.pallas import tpu_sc as plsc

N = 10000
E = 160000
D = 256
NPAD = 10240
NC = 2
NS = 16
ROWS_PER_TILE = NPAD // NS
EDGES_PER_TILE = E // NS
CHUNK = 32
IDXROW = 128
NROW = 80
EPAD_TILE = NROW * IDXROW
NCHUNK = EPAD_TILE // CHUNK
NB = 8
ROUNDS = NCHUNK // NB
EMB_CHUNK = 80
EMB_NCHUNK = 4
GRID = NPAD // 256

_f32 = jnp.float32


def _emb_deg_body(nid_hbm, srcp_hbm, dstp_hbm, emb_hbm, ones_hbm, zeros1_hbm,
             h0_out, degout_out, degin_out,
             nidx_v, rows_v, eidx_v, ones_v, deg_sh, sem):
  c = lax.axis_index("c")
  s = lax.axis_index("s")
  w = c * NS + s

  pltpu.sync_copy(zeros1_hbm, deg_sh.at[pl.ds(s * ROWS_PER_TILE, ROWS_PER_TILE)])
  pltpu.sync_copy(ones_hbm, ones_v)

  pltpu.sync_copy(nid_hbm.at[w], nidx_v)

  def emb_chunk(j, carry):
    pltpu.async_copy(emb_hbm.at[nidx_v.at[j]], rows_v, sem).wait()
    base = pl.multiple_of(w * (EMB_NCHUNK * EMB_CHUNK) + j * EMB_CHUNK, 8)
    pltpu.sync_copy(rows_v, h0_out.at[pl.ds(base, EMB_CHUNK)])
    return carry

  lax.fori_loop(0, EMB_NCHUNK, emb_chunk, 0)

  plsc.subcore_barrier()

  def deg_pass(edges_hbm):
    pltpu.sync_copy(edges_hbm.at[s], eidx_v)

    def deg_chunk(j, carry):
      pltpu.sync_copy(ones_v, deg_sh.at[eidx_v.at[j]], add=True)
      return carry

    lax.fori_loop(0, NROW, deg_chunk, 0)

  @pl.when(c == 0)
  def _():
    deg_pass(srcp_hbm)

  @pl.when(c == 1)
  def _():
    deg_pass(dstp_hbm)

  plsc.subcore_barrier()

  sl = pl.ds(s * ROWS_PER_TILE, ROWS_PER_TILE)

  @pl.when(c == 0)
  def _():
    pltpu.sync_copy(deg_sh.at[sl], degout_out.at[sl])

  @pl.when(c == 1)
  def _():
    pltpu.sync_copy(deg_sh.at[sl], degin_out.at[sl])


def _mp_body(gl_hbm, gr_hbm, srcp_hbm, dstp_hbm, zeros_hbm,
             outl, outr, sidx_v, didx_v, rows_v, acc_sh, *sems):
  gs = list(sems[:NB])
  ss = list(sems[NB:])
  c = lax.axis_index("c")
  s = lax.axis_index("s")
  sl = pl.ds(s * ROWS_PER_TILE, ROWS_PER_TILE)

  pltpu.sync_copy(zeros_hbm, acc_sh.at[sl])
  pltpu.sync_copy(srcp_hbm.at[s], sidx_v)
  plsc.subcore_barrier()

  def run(g_hbm):
    per_row = IDXROW // CHUNK

    def idx_slice(idx_v, r, b):
      return idx_v.at[(NB // per_row) * r + b // per_row,
                      pl.ds((b % per_row) * CHUNK, CHUNK)]

    def gbuf(b):
      return rows_v.at[pl.ds(b * CHUNK, CHUNK)]

    def fire_gather(b, r):
      pltpu.async_copy(g_hbm.at[idx_slice(sidx_v, r, b)], gbuf(b), gs[b])

    def fire_scatter(b, rl):
      pltpu.async_copy(gbuf(b), acc_sh.at[idx_slice(didx_v, rl, b)],
                       ss[b], add=True)

    def wait_gather(b):
      pltpu.make_async_copy(g_hbm.at[pl.ds(0, CHUNK)], gbuf(b),
                            gs[b]).wait()

    def wait_scatter(b):
      pltpu.make_async_copy(gbuf(b), acc_sh.at[pl.ds(0, CHUNK)],
                            ss[b]).wait()

    rh = ROUNDS // 2
    for h in range(2):
      pltpu.sync_copy(dstp_hbm.at[s, pl.ds(h * (NROW // 2), NROW // 2)],
                      didx_v)
      for b in range(NB):
        fire_gather(b, h * rh)

      def ring(rl, carry):
        for b in range(NB):
          wait_gather(b)
          fire_scatter(b, rl)

        @pl.when(rl < rh - 1)
        def _():
          for b in range(NB):
            wait_scatter(b)
            fire_gather(b, h * rh + rl + 1)

        return carry

      lax.fori_loop(0, rh, ring, 0)
      for b in range(NB):
        wait_scatter(b)

  @pl.when(c == 0)
  def _():
    run(gl_hbm)

  @pl.when(c == 1)
  def _():
    run(gr_hbm)

  plsc.subcore_barrier()

  @pl.when(c == 0)
  def _():
    pltpu.sync_copy(acc_sh.at[sl], outl.at[sl])

  @pl.when(c == 1)
  def _():
    pltpu.sync_copy(acc_sh.at[sl], outr.at[sl])


def _pre_body(h0_ref, doutb_ref, gl_ref, gr_ref):
  deg = doutb_ref[...]
  norm = jnp.where(deg > 0, lax.rsqrt(deg), 0.0)
  h = h0_ref[...]
  gl_ref[...] = h[:, :128] * norm
  gr_ref[...] = h[:, 128:] * norm


def _mm_body(al_ref, ar_ref, dinb_ref, doutb_ref, w_ref, b_ref,
             gl_ref, gr_ref):
  din = dinb_ref[...]
  nin = jnp.where(din > 0, lax.rsqrt(din), 0.0)
  x = jnp.concatenate([al_ref[...] * nin, ar_ref[...] * nin], axis=1)
  h = jnp.dot(x, w_ref[...], preferred_element_type=jnp.float32) + b_ref[...]
  dout = doutb_ref[...]
  nout = jnp.where(dout > 0, lax.rsqrt(dout), 0.0)
  gl_ref[...] = h[:, :128] * nout
  gr_ref[...] = h[:, 128:] * nout


def _mm2_body(al_ref, ar_ref, dinb_ref, w_ref, b_ref, lng_ref, lnb_ref,
              lw_ref, lb_ref, out_ref, max_scr):
  n = pl.program_id(0)
  din = dinb_ref[...]
  nin = jnp.where(din > 0, lax.rsqrt(din), 0.0)
  x = jnp.concatenate([al_ref[...] * nin, ar_ref[...] * nin], axis=1)
  h = jnp.dot(x, w_ref[...], preferred_element_type=jnp.float32) + b_ref[...]
  rid = n * 256 + lax.broadcasted_iota(jnp.int32, (256, D), 0)
  hm = jnp.where(rid < N, h, -jnp.inf)
  bmax = jnp.max(hm, axis=0, keepdims=True)

  @pl.when(n == 0)
  def _():
    max_scr[...] = bmax

  @pl.when(n > 0)
  def _():
    max_scr[...] = jnp.maximum(max_scr[...], bmax)

  @pl.when(n == GRID - 1)
  def _():
    pooled = max_scr[...]
    mu = jnp.mean(pooled)
    var = jnp.mean((pooled - mu) ** 2)
    xn = (pooled - mu) * lax.rsqrt(var + 1e-5) * lng_ref[...] + lnb_ref[...]
    o = jnp.dot(xn, lw_ref[...], preferred_element_type=jnp.float32)
    out_ref[...] = jnp.maximum(o + lb_ref[...], 0.0)


def _block(n128):
  return pl.BlockSpec((256, n128), lambda n: (n, 0))


def _full(shape):
  return pl.BlockSpec(shape, lambda n: tuple(0 for _ in shape))


_pre_call = pl.pallas_call(
    _pre_body,
    grid=(GRID,),
    in_specs=[_block(D), _block(128)],
    out_specs=[_block(128), _block(128)],
    out_shape=[jax.ShapeDtypeStruct((NPAD, 128), _f32)] * 2,
)

_mm1_call = pl.pallas_call(
    _mm_body,
    grid=(GRID,),
    in_specs=[_block(128), _block(128), _block(128), _block(128),
              _full((D, D)), _full((1, D))],
    out_specs=[_block(128), _block(128)],
    out_shape=[jax.ShapeDtypeStruct((NPAD, 128), _f32)] * 2,
)

_mm2_call = pl.pallas_call(
    _mm2_body,
    grid=(GRID,),
    in_specs=[_block(128), _block(128), _block(128),
              _full((D, D)), _full((1, D)), _full((1, D)), _full((1, D)),
              _full((D, 32)), _full((1, 32))],
    out_specs=pl.BlockSpec((1, 32), lambda n: (0, 0)),
    out_shape=jax.ShapeDtypeStruct((1, 32), _f32),
    scratch_shapes=[pltpu.VMEM((1, D), _f32)],
)


@functools.lru_cache(maxsize=1)
def _sc_kernels():
  mesh = plsc.VectorSubcoreMesh(core_axis_name="c", subcore_axis_name="s",
                                num_cores=NC, num_subcores=NS)
  emb_deg = pl.kernel(
      _emb_deg_body,
      out_type=[
          jax.ShapeDtypeStruct((NPAD, D), _f32),
          jax.ShapeDtypeStruct((NPAD,), _f32),
          jax.ShapeDtypeStruct((NPAD,), _f32),
      ],
      mesh=mesh,
      scratch_types=[
          pltpu.VMEM((EMB_NCHUNK, EMB_CHUNK), jnp.int32),
          pltpu.VMEM((EMB_CHUNK, D), _f32),
          pltpu.VMEM((NROW, IDXROW), jnp.int32),
          pltpu.VMEM((IDXROW,), _f32),
          pltpu.VMEM_SHARED((NPAD,), _f32),
          pltpu.SemaphoreType.DMA,
      ],
  )
  mp = pl.kernel(
      _mp_body,
      out_type=[
          jax.ShapeDtypeStruct((NPAD, 128), _f32),
          jax.ShapeDtypeStruct((NPAD, 128), _f32),
      ],
      mesh=mesh,
      scratch_types=[
          pltpu.VMEM((NROW, IDXROW), jnp.int32),
          pltpu.VMEM((NROW // 2, IDXROW), jnp.int32),
          pltpu.VMEM((NB * CHUNK, 128), _f32),
          pltpu.VMEM_SHARED((NPAD, 128), _f32),
      ] + [pltpu.SemaphoreType.DMA] * (2 * NB),
  )
  return emb_deg, mp


def _pad_edges(e):
  e = e.reshape(NS, EDGES_PER_TILE)
  pad = N + (jnp.arange(EPAD_TILE - EDGES_PER_TILE, dtype=jnp.int32) % (NPAD - N))
  pad = jnp.broadcast_to(pad, (NS, EPAD_TILE - EDGES_PER_TILE))
  return jnp.concatenate([e, pad], axis=1).reshape(NS, NROW, IDXROW)


def kernel(node_ids, edge_index, emb_table, W1, b1, W2, b2,
           ln_gamma, ln_beta, lin_W, lin_b):
  node_ids = node_ids.astype(jnp.int32)
  src = edge_index[0].astype(jnp.int32)
  dst = edge_index[1].astype(jnp.int32)

  nid_pad = jnp.concatenate(
      [node_ids, jnp.zeros((NPAD - N,), jnp.int32)]
  ).reshape(NC * NS, EMB_NCHUNK, EMB_CHUNK)
  srcp = _pad_edges(src)
  dstp = _pad_edges(dst)

  ones_c = jnp.ones((IDXROW,), _f32)
  zeros1 = jnp.zeros((ROWS_PER_TILE,), _f32)
  zeros2 = jnp.zeros((ROWS_PER_TILE, 128), _f32)

  emb_deg, mp = _sc_kernels()
  h0, deg_out, deg_in = emb_deg(nid_pad, srcp, dstp, emb_table,
                                ones_c, zeros1)

  dout_b = jnp.broadcast_to(deg_out[:, None], (NPAD, 128))
  din_b = jnp.broadcast_to(deg_in[:, None], (NPAD, 128))

  g0l, g0r = _pre_call(h0, dout_b)
  a1l, a1r = mp(g0l, g0r, srcp, dstp, zeros2)
  g1l, g1r = _mm1_call(a1l, a1r, din_b, dout_b, W1, b1.reshape(1, D))
  a2l, a2r = mp(g1l, g1r, srcp, dstp, zeros2)
  out = _mm2_call(a2l, a2r, din_b, W2, b2.reshape(1, D),
                  ln_gamma.reshape(1, D), ln_beta.reshape(1, D),
                  lin_W, lin_b.reshape(1, 32))
  return out.reshape(32)


# speedup vs baseline: 1.2243x; 1.0483x over previous
import functools

import jax
import jax.numpy as jnp
from jax import lax
from jax.experimental import pallas as pl
from jax.experimental.pallas import tpu as pltpu
from jax.experimental.pallas import tpu_sc as plsc

N = 10000
E = 160000
D = 256
NPAD = 10240
NC = 2
NS = 16
ROWS_PER_TILE = NPAD // NS
EDGES_PER_TILE = E // NS
CHUNK = 32
IDXROW = 128
NROW = 80
EPAD_TILE = NROW * IDXROW
NCHUNK = EPAD_TILE // CHUNK
NB = 8
ROUNDS = NCHUNK // NB
EMB_CHUNK = 80
EMB_NCHUNK = 4
GRID = NPAD // 256

_f32 = jnp.float32


def _emb_deg_body(nid_hbm, srcp_hbm, dstp_hbm, emb_hbm, ones_hbm, zeros1_hbm,
             h0_out, degout_out, degin_out,
             nidx_v, rows_v, eidx_v, ones_v, deg_sh, dsem, *esems):
  c = lax.axis_index("c")
  s = lax.axis_index("s")
  w = c * NS + s

  pltpu.sync_copy(zeros1_hbm, deg_sh.at[pl.ds(s * ROWS_PER_TILE, ROWS_PER_TILE)])
  pltpu.sync_copy(ones_hbm, ones_v)
  pltpu.sync_copy(nid_hbm.at[w], nidx_v)
  plsc.subcore_barrier()

  def deg_fire(edges_hbm):
    pltpu.sync_copy(edges_hbm.at[s], eidx_v)

    def deg_chunk(j, carry):
      pltpu.async_copy(ones_v, deg_sh.at[eidx_v.at[j]], dsem, add=True)
      return carry

    lax.fori_loop(0, NROW, deg_chunk, 0)

  @pl.when(c == 0)
  def _():
    deg_fire(srcp_hbm)

  @pl.when(c == 1)
  def _():
    deg_fire(dstp_hbm)

  def ebuf(j):
    return rows_v.at[pl.ds(j * EMB_CHUNK, EMB_CHUNK)]

  for j in range(EMB_NCHUNK):
    pltpu.async_copy(emb_hbm.at[nidx_v.at[j]], ebuf(j), esems[j])
  for j in range(EMB_NCHUNK):
    base = w * (EMB_NCHUNK * EMB_CHUNK) + j * EMB_CHUNK
    pltpu.make_async_copy(emb_hbm.at[pl.ds(0, EMB_CHUNK)], ebuf(j),
                          esems[j]).wait()
    pltpu.async_copy(ebuf(j), h0_out.at[pl.ds(base, EMB_CHUNK)], esems[j])
  for j in range(EMB_NCHUNK):
    pltpu.make_async_copy(ebuf(j), h0_out.at[pl.ds(0, EMB_CHUNK)],
                          esems[j]).wait()

  def deg_drain(j, carry):
    pltpu.make_async_copy(ones_v, deg_sh.at[pl.ds(0, IDXROW)], dsem).wait()
    return carry

  lax.fori_loop(0, NROW, deg_drain, 0)

  plsc.subcore_barrier()

  sl = pl.ds(s * ROWS_PER_TILE, ROWS_PER_TILE)

  @pl.when(c == 0)
  def _():
    pltpu.sync_copy(deg_sh.at[sl], degout_out.at[sl])

  @pl.when(c == 1)
  def _():
    pltpu.sync_copy(deg_sh.at[sl], degin_out.at[sl])


def _mp_body(gl_hbm, gr_hbm, srcp_hbm, dstp_hbm, zeros_hbm,
             outl, outr, sidx_v, didx_v, rows_v, acc_sh, *sems):
  gs = list(sems[:NB])
  ss = list(sems[NB:])
  c = lax.axis_index("c")
  s = lax.axis_index("s")
  sl = pl.ds(s * ROWS_PER_TILE, ROWS_PER_TILE)

  pltpu.sync_copy(zeros_hbm, acc_sh.at[sl])
  pltpu.sync_copy(srcp_hbm.at[s], sidx_v)
  plsc.subcore_barrier()

  def run(g_hbm):
    per_row = IDXROW // CHUNK

    def idx_slice(idx_v, r, b):
      return idx_v.at[(NB // per_row) * r + b // per_row,
                      pl.ds((b % per_row) * CHUNK, CHUNK)]

    def gbuf(b):
      return rows_v.at[pl.ds(b * CHUNK, CHUNK)]

    def fire_gather(b, r):
      pltpu.async_copy(g_hbm.at[idx_slice(sidx_v, r, b)], gbuf(b), gs[b])

    def fire_scatter(b, rl):
      pltpu.async_copy(gbuf(b), acc_sh.at[idx_slice(didx_v, rl, b)],
                       ss[b], add=True)

    def wait_gather(b):
      pltpu.make_async_copy(g_hbm.at[pl.ds(0, CHUNK)], gbuf(b),
                            gs[b]).wait()

    def wait_scatter(b):
      pltpu.make_async_copy(gbuf(b), acc_sh.at[pl.ds(0, CHUNK)],
                            ss[b]).wait()

    rh = ROUNDS // 2
    for h in range(2):
      pltpu.sync_copy(dstp_hbm.at[s, pl.ds(h * (NROW // 2), NROW // 2)],
                      didx_v)
      for b in range(NB):
        fire_gather(b, h * rh)

      def ring(rl, carry):
        for b in range(NB):
          wait_gather(b)
          fire_scatter(b, rl)

        @pl.when(rl < rh - 1)
        def _():
          for b in range(NB):
            wait_scatter(b)
            fire_gather(b, h * rh + rl + 1)

        return carry

      lax.fori_loop(0, rh, ring, 0)
      for b in range(NB):
        wait_scatter(b)

  @pl.when(c == 0)
  def _():
    run(gl_hbm)

  @pl.when(c == 1)
  def _():
    run(gr_hbm)

  plsc.subcore_barrier()

  @pl.when(c == 0)
  def _():
    pltpu.sync_copy(acc_sh.at[sl], outl.at[sl])

  @pl.when(c == 1)
  def _():
    pltpu.sync_copy(acc_sh.at[sl], outr.at[sl])


def _norm_col(dref):
  d = dref[0, 0, :]
  return jnp.where(d > 0, lax.rsqrt(d), 0.0)[:, None]


def _pre_body(h0_ref, doutb_ref, gl_ref, gr_ref):
  norm = _norm_col(doutb_ref)
  h = h0_ref[...]
  gl_ref[...] = h[:, :128] * norm
  gr_ref[...] = h[:, 128:] * norm


def _mm_body(al_ref, ar_ref, dinb_ref, doutb_ref, w_ref, b_ref,
             gl_ref, gr_ref):
  nin = _norm_col(dinb_ref)
  x = jnp.concatenate([al_ref[...] * nin, ar_ref[...] * nin], axis=1)
  h = jnp.dot(x, w_ref[...], preferred_element_type=jnp.float32) + b_ref[...]
  nout = _norm_col(doutb_ref)
  gl_ref[...] = h[:, :128] * nout
  gr_ref[...] = h[:, 128:] * nout


def _mm2_body(al_ref, ar_ref, dinb_ref, w_ref, b_ref, lng_ref, lnb_ref,
              lw_ref, lb_ref, out_ref, max_scr):
  n = pl.program_id(0)
  nin = _norm_col(dinb_ref)
  x = jnp.concatenate([al_ref[...] * nin, ar_ref[...] * nin], axis=1)
  h = jnp.dot(x, w_ref[...], preferred_element_type=jnp.float32) + b_ref[...]
  rid = n * 256 + lax.broadcasted_iota(jnp.int32, (256, D), 0)
  hm = jnp.where(rid < N, h, -jnp.inf)
  bmax = jnp.max(hm, axis=0, keepdims=True)

  @pl.when(n == 0)
  def _():
    max_scr[...] = bmax

  @pl.when(n > 0)
  def _():
    max_scr[...] = jnp.maximum(max_scr[...], bmax)

  @pl.when(n == GRID - 1)
  def _():
    pooled = max_scr[...]
    mu = jnp.mean(pooled)
    var = jnp.mean((pooled - mu) ** 2)
    xn = (pooled - mu) * lax.rsqrt(var + 1e-5) * lng_ref[...] + lnb_ref[...]
    o = jnp.dot(xn, lw_ref[...], preferred_element_type=jnp.float32)
    out_ref[...] = jnp.maximum(o + lb_ref[...], 0.0)


def _block(n128):
  return pl.BlockSpec((256, n128), lambda n: (n, 0))


def _full(shape):
  return pl.BlockSpec(shape, lambda n: tuple(0 for _ in shape))


_dblock = pl.BlockSpec((1, 1, D), lambda n: (n, 0, 0))

_pre_call = pl.pallas_call(
    _pre_body,
    grid=(GRID,),
    in_specs=[_block(D), _dblock],
    out_specs=[_block(128), _block(128)],
    out_shape=[jax.ShapeDtypeStruct((NPAD, 128), _f32)] * 2,
)

_mm1_call = pl.pallas_call(
    _mm_body,
    grid=(GRID,),
    in_specs=[_block(128), _block(128), _dblock, _dblock,
              _full((D, D)), _full((1, D))],
    out_specs=[_block(128), _block(128)],
    out_shape=[jax.ShapeDtypeStruct((NPAD, 128), _f32)] * 2,
)

_mm2_call = pl.pallas_call(
    _mm2_body,
    grid=(GRID,),
    in_specs=[_block(128), _block(128), _dblock,
              _full((D, D)), _full((1, D)), _full((1, D)), _full((1, D)),
              _full((D, 32)), _full((1, 32))],
    out_specs=pl.BlockSpec((1, 32), lambda n: (0, 0)),
    out_shape=jax.ShapeDtypeStruct((1, 32), _f32),
    scratch_shapes=[pltpu.VMEM((1, D), _f32)],
)


@functools.lru_cache(maxsize=1)
def _sc_kernels():
  mesh = plsc.VectorSubcoreMesh(core_axis_name="c", subcore_axis_name="s",
                                num_cores=NC, num_subcores=NS)
  emb_deg = pl.kernel(
      _emb_deg_body,
      out_type=[
          jax.ShapeDtypeStruct((NPAD, D), _f32),
          jax.ShapeDtypeStruct((NPAD,), _f32),
          jax.ShapeDtypeStruct((NPAD,), _f32),
      ],
      mesh=mesh,
      scratch_types=[
          pltpu.VMEM((EMB_NCHUNK, EMB_CHUNK), jnp.int32),
          pltpu.VMEM((EMB_NCHUNK * EMB_CHUNK, D), _f32),
          pltpu.VMEM((NROW, IDXROW), jnp.int32),
          pltpu.VMEM((IDXROW,), _f32),
          pltpu.VMEM_SHARED((NPAD,), _f32),
      ] + [pltpu.SemaphoreType.DMA] * (1 + EMB_NCHUNK),
  )
  mp = pl.kernel(
      _mp_body,
      out_type=[
          jax.ShapeDtypeStruct((NPAD, 128), _f32),
          jax.ShapeDtypeStruct((NPAD, 128), _f32),
      ],
      mesh=mesh,
      scratch_types=[
          pltpu.VMEM((NROW, IDXROW), jnp.int32),
          pltpu.VMEM((NROW // 2, IDXROW), jnp.int32),
          pltpu.VMEM((NB * CHUNK, 128), _f32),
          pltpu.VMEM_SHARED((NPAD, 128), _f32),
      ] + [pltpu.SemaphoreType.DMA] * (2 * NB),
  )
  return emb_deg, mp


def _pad_edges(e):
  e = e.reshape(NS, EDGES_PER_TILE)
  pad = N + (jnp.arange(EPAD_TILE - EDGES_PER_TILE, dtype=jnp.int32) % (NPAD - N))
  pad = jnp.broadcast_to(pad, (NS, EPAD_TILE - EDGES_PER_TILE))
  return jnp.concatenate([e, pad], axis=1).reshape(NS, NROW, IDXROW)


def kernel(node_ids, edge_index, emb_table, W1, b1, W2, b2,
           ln_gamma, ln_beta, lin_W, lin_b):
  node_ids = node_ids.astype(jnp.int32)
  src = edge_index[0].astype(jnp.int32)
  dst = edge_index[1].astype(jnp.int32)

  nid_pad = jnp.concatenate(
      [node_ids, jnp.zeros((NPAD - N,), jnp.int32)]
  ).reshape(NC * NS, EMB_NCHUNK, EMB_CHUNK)
  srcp = _pad_edges(src)
  dstp = _pad_edges(dst)

  ones_c = jnp.ones((IDXROW,), _f32)
  zeros1 = jnp.zeros((ROWS_PER_TILE,), _f32)
  zeros2 = jnp.zeros((ROWS_PER_TILE, 128), _f32)

  emb_deg, mp = _sc_kernels()
  h0, deg_out, deg_in = emb_deg(nid_pad, srcp, dstp, emb_table,
                                ones_c, zeros1)

  dout_b = deg_out.reshape(GRID, 1, D)
  din_b = deg_in.reshape(GRID, 1, D)

  g0l, g0r = _pre_call(h0, dout_b)
  a1l, a1r = mp(g0l, g0r, srcp, dstp, zeros2)
  g1l, g1r = _mm1_call(a1l, a1r, din_b, dout_b, W1, b1.reshape(1, D))
  a2l, a2r = mp(g1l, g1r, srcp, dstp, zeros2)
  out = _mm2_call(a2l, a2r, din_b, W2, b2.reshape(1, D),
                  ln_gamma.reshape(1, D), ln_beta.reshape(1, D),
                  lin_W, lin_b.reshape(1, 32))
  return out.reshape(32)
